# baseline (speedup 1.0000x reference)
import functools

import jax
import jax.numpy as jnp
from jax import lax
from jax.experimental import pallas as pl
from jax.experimental.pallas import tpu as pltpu
from jax.experimental.pallas import tpu_sc as plsc

D = 128
V = 1000
VP = 1024
B = 4096 * 200
NC, NS = 2, 16
NW = NC * NS
BPW = B // NW
IW = 128
SUB = 2
CH = IW * SUB
NCH = BPW // CH
NB = 2
NGRP = NCH // NB
IDX_ROWS = BPW // IW

_mesh = plsc.VectorSubcoreMesh(core_axis_name="c", subcore_axis_name="s")


@functools.partial(
    pl.kernel,
    mesh=_mesh,
    out_type=jax.ShapeDtypeStruct((B, D), jnp.float32),
    scratch_types=[
        pltpu.VMEM_SHARED((VP, D), jnp.float32),
        pltpu.VMEM((IDX_ROWS, IW), jnp.int32),
        pltpu.VMEM((NB * CH, D), jnp.float32),
        [pltpu.SemaphoreType.DMA] * NB,
        [pltpu.SemaphoreType.DMA] * NB,
    ],
)
def _emb_lookup(idx_hbm, table_hbm, out_hbm, table_sh, idx_v, rows_v, gsems, wsems):
    cid = lax.axis_index("c")
    sid = lax.axis_index("s")
    wid = sid * NC + cid
    base = wid * BPW

    rows_per_sub = VP // NS
    pltpu.sync_copy(
        table_hbm.at[pl.ds(sid * rows_per_sub, rows_per_sub)],
        table_sh.at[pl.ds(sid * rows_per_sub, rows_per_sub)],
    )
    pltpu.sync_copy(idx_hbm.at[pl.ds(wid * IDX_ROWS, IDX_ROWS)], idx_v)
    plsc.subcore_barrier()

    def _g(chunk, b, k):
        return pltpu.make_async_copy(
            table_sh.at[idx_v.at[chunk * SUB + k]],
            rows_v.at[pl.ds(b * CH + k * IW, IW)],
            gsems[b],
        )

    def _w(chunk, b):
        return pltpu.make_async_copy(
            rows_v.at[pl.ds(b * CH, CH)],
            out_hbm.at[pl.ds(base + chunk * CH, CH)],
            wsems[b],
        )

    for b in range(NB):
        for k in range(SUB):
            _g(b, b, k).start()

    def body(grp, _):
        c0 = grp * NB
        for b in range(NB):
            for k in range(SUB):
                _g(c0 + b, b, k).wait()
            _w(c0 + b, b).start()
        for b in range(NB):
            @pl.when(c0 + b + NB < NCH)
            def _():
                _w(c0 + b, b).wait()
                for k in range(SUB):
                    _g(c0 + b + NB, b, k).start()

        return 0

    lax.fori_loop(0, NGRP, body, 0)

    for b in range(NB):
        _w((NGRP - 1) * NB + b, b).wait()


def kernel(time, emb_weight):
    idx = time.reshape(-1, IW).astype(jnp.int32)
    table = jnp.zeros((VP, D), jnp.float32).at[:V].set(emb_weight)
    out = _emb_lookup(idx, table)
    return out.reshape(time.shape + (D,))


# speedup vs baseline: 1.4754x; 1.4754x over previous
import functools

import jax
import jax.numpy as jnp
from jax import lax
from jax.experimental import pallas as pl
from jax.experimental.pallas import tpu as pltpu
from jax.experimental.pallas import tpu_sc as plsc

D = 128
V = 1000
VP = 1024
B = 4096 * 200
NC, NS = 2, 16
NW = NC * NS
BPW = B // NW
CH = 128
NCH = BPW // CH
SS = 4
NGRP = NCH // SS

_mesh = plsc.VectorSubcoreMesh(core_axis_name="c", subcore_axis_name="s")


@functools.partial(
    pl.kernel,
    mesh=_mesh,
    out_type=jax.ShapeDtypeStruct((B, D), jnp.float32),
    scratch_types=[
        pltpu.VMEM_SHARED((VP, D), jnp.float32),
        pltpu.VMEM((NCH, CH), jnp.int32),
        pltpu.VMEM((SS * CH, D), jnp.float32),
        [pltpu.SemaphoreType.DMA] * SS,
        [pltpu.SemaphoreType.DMA] * SS,
    ],
)
def _emb_lookup(idx_hbm, table_hbm, out_hbm, table_sh, idx_v, rows_v, gsems, wsems):
    cid = lax.axis_index("c")
    sid = lax.axis_index("s")
    wid = sid * NC + cid
    base = wid * BPW

    rows_per_sub = VP // NS
    pltpu.sync_copy(
        table_hbm.at[pl.ds(sid * rows_per_sub, rows_per_sub)],
        table_sh.at[pl.ds(sid * rows_per_sub, rows_per_sub)],
    )
    pltpu.sync_copy(idx_hbm.at[pl.ds(wid * NCH, NCH)], idx_v)
    plsc.subcore_barrier()

    def _g(chunk, b):
        return pltpu.make_async_copy(
            table_sh.at[idx_v.at[chunk]],
            rows_v.at[pl.ds(b * CH, CH)],
            gsems[b],
        )

    def _w(chunk, b):
        return pltpu.make_async_copy(
            rows_v.at[pl.ds(b * CH, CH)],
            out_hbm.at[pl.ds(base + chunk * CH, CH)],
            wsems[b],
        )

    _g(0, 0).start()
    _g(1, 1).start()

    def body(grp, _):
        c0 = grp * SS

        @pl.when(grp > 0)
        def _():
            _w(c0 - 2, 2).wait()
            _w(c0 - 1, 3).wait()

        _g(c0 + 2, 2).start()
        _g(c0 + 3, 3).start()
        _g(c0, 0).wait()
        _g(c0 + 1, 1).wait()
        _w(c0, 0).start()
        _w(c0 + 1, 1).start()
        _w(c0, 0).wait()
        _w(c0 + 1, 1).wait()

        @pl.when(c0 + SS < NCH)
        def _():
            _g(c0 + 4, 0).start()
            _g(c0 + 5, 1).start()

        _g(c0 + 2, 2).wait()
        _g(c0 + 3, 3).wait()
        _w(c0 + 2, 2).start()
        _w(c0 + 3, 3).start()
        return 0

    lax.fori_loop(0, NGRP, body, 0)

    _w(NCH - 2, 2).wait()
    _w(NCH - 1, 3).wait()


def kernel(time, emb_weight):
    idx = time.reshape(-1, CH).astype(jnp.int32)
    table = jnp.zeros((VP, D), jnp.float32).at[:V].set(emb_weight)
    out = _emb_lookup(idx, table)
    return out.reshape(time.shape + (D,))
